# baseline (speedup 1.0000x reference)
import jax
import jax.numpy as jnp
from jax import lax
from jax.experimental import pallas as pl
from jax.experimental.pallas import tpu as pltpu
from jax.experimental.pallas import tpu_sc as plsc

N = 10000
E = 320000
D_IN = 128
D_HID = 64
D_PAD = 16

NC = 2
NS = 16
NW = NC * NS
B = 128
NB0 = 160
NB1 = 0
NBMAX = max(NB0, NB1)
NBT = NS * (NB0 + NB1)
E_PAD = NBT * B
N_ACC = 10112
ROWS_PER_TILE = N_ACC // NS


def _scatter_rows_kernel(D, nb0, nb1, with_gather):
  mesh = plsc.VectorSubcoreMesh(core_axis_name="c", subcore_axis_name="s")
  nbmax = max(nb0, nb1)
  scratch = [
      pltpu.VMEM((nbmax, B), jnp.int32),
      pltpu.VMEM((B, D), jnp.float32),
      pltpu.VMEM_SHARED((N_ACC, D), jnp.float32),
      pltpu.SemaphoreType.DMA,
  ]
  if with_gather:
    scratch += [
        pltpu.VMEM((nbmax, B), jnp.int32),
        pltpu.VMEM((B, D), jnp.float32),
        pltpu.SemaphoreType.DMA,
    ]

  def body(table, srcf, dstf, out, *refs):
    if with_gather:
      dst_v, rows, acc, sem, src_v, rows1, sem1 = refs
    else:
      dst_v, rows, acc, sem = refs
    cid = lax.axis_index("c")
    sid = lax.axis_index("s")
    nb = lax.select(cid == 0, jnp.int32(nb0), jnp.int32(nb1))
    start_b = lax.select(cid == 0, sid * nb0, NS * nb0 + sid * nb1)

    def fill_rows(val16):
      def frow(i, carry):
        for j in range(D // 16):
          rows[i, pl.ds(j * 16, 16)] = val16
        return carry

      lax.fori_loop(0, B, frow, 0)

    fill_rows(jnp.zeros((16,), jnp.float32))
    base = sid * ROWS_PER_TILE
    for k in range(ROWS_PER_TILE // B):
      pltpu.sync_copy(rows, acc.at[pl.ds(base + k * B, B)])
    rem_rows = ROWS_PER_TILE % B
    if rem_rows:
      pltpu.sync_copy(rows.at[pl.ds(0, rem_rows)],
                      acc.at[pl.ds(base + (ROWS_PER_TILE // B) * B,
                                   rem_rows)])
    if with_gather:
      pltpu.sync_copy(srcf.at[pl.ds(start_b, nbmax)], src_v)
    else:
      fill_rows(jnp.ones((16,), jnp.float32))
    pltpu.sync_copy(dstf.at[pl.ds(start_b, nbmax)], dst_v)
    plsc.subcore_barrier()

    if with_gather:
      pltpu.async_copy(table.at[src_v.at[0]], rows, sem)

      def pair(t, carry):
        j0 = 2 * t
        j1 = j0 + 1
        pltpu.make_async_copy(table.at[src_v.at[j0]], rows, sem).wait()
        pltpu.async_copy(table.at[src_v.at[j1]], rows1, sem1)
        pltpu.sync_copy(rows, acc.at[dst_v.at[j0]], add=True)
        jn = lax.rem(j0 + 2, nb)
        pltpu.async_copy(table.at[src_v.at[jn]], rows, sem)
        pltpu.make_async_copy(table.at[src_v.at[j1]], rows1, sem1).wait()
        pltpu.sync_copy(rows1, acc.at[dst_v.at[j1]], add=True)
        return carry

      lax.fori_loop(0, nb // 2, pair, 0)
      pltpu.make_async_copy(table.at[src_v.at[0]], rows, sem).wait()
    else:

      def edge_batch(j, carry):
        pltpu.sync_copy(rows, acc.at[dst_v.at[j]], add=True)
        return carry

      lax.fori_loop(0, nb, edge_batch, 0)
    plsc.subcore_barrier()
    pltpu.sync_copy(
        acc.at[pl.ds(sid * ROWS_PER_TILE, ROWS_PER_TILE)],
        out.at[cid, pl.ds(sid * ROWS_PER_TILE, ROWS_PER_TILE)])

  return pl.kernel(
      body,
      out_type=jax.ShapeDtypeStruct((NC, N_ACC, D), jnp.float32),
      mesh=mesh,
      scratch_types=scratch,
      compiler_params=pltpu.CompilerParams(use_tc_tiling_on_sc=False),
  )


def _tc1_body(x_ref, w1_ref, dega_ref, hs1_ref, dis_ref):
  deg = dega_ref[0, :, 0:1] + dega_ref[1, :, 0:1] + 1.0
  dis = lax.rsqrt(deg)
  t = jnp.dot(x_ref[...], w1_ref[...], preferred_element_type=jnp.float32)
  hs1_ref[...] = t * dis
  dis_ref[...] = jnp.broadcast_to(dis, dis_ref.shape)


def _tc_mid_body(acc_ref, hs_ref, dis_ref, w_ref, b_ref, out_ref):
  dis = dis_ref[:, 0:1]
  s = acc_ref[0] + acc_ref[1] + hs_ref[...]
  h = jnp.maximum(dis * s + b_ref[...], 0.0)
  out_ref[...] = jnp.dot(h, w_ref[...],
                         preferred_element_type=jnp.float32) * dis


def _tc4_body(acc_ref, hs_ref, dis_ref, b_ref, out_ref):
  dis = dis_ref[:, 0:1]
  o = dis * (acc_ref[0] + acc_ref[1] + hs_ref[...]) + b_ref[...]
  o = o[:, :6]
  m = jnp.max(o, axis=1, keepdims=True)
  e = jnp.exp(o - m)
  lse = jnp.log(jnp.sum(e, axis=1, keepdims=True))
  out_ref[...] = o - m - lse


_R = 1000
_GRID = N // _R


def _row_spec(d):
  return pl.BlockSpec((_R, d), lambda i: (i, 0))


def _acc_spec(d):
  return pl.BlockSpec((NC, _R, d), lambda i: (0, i, 0))


def _full_spec(a, b):
  return pl.BlockSpec((a, b), lambda i: (0, 0))


_tc1 = pl.pallas_call(
    _tc1_body,
    grid=(_GRID,),
    in_specs=[_row_spec(D_IN), _full_spec(D_IN, D_HID), _acc_spec(D_PAD)],
    out_specs=[_row_spec(D_HID), _row_spec(8)],
    out_shape=[jax.ShapeDtypeStruct((N, D_HID), jnp.float32),
               jax.ShapeDtypeStruct((N, 8), jnp.float32)],
)

_tc_mid = pl.pallas_call(
    _tc_mid_body,
    grid=(_GRID,),
    in_specs=[_acc_spec(D_HID), _row_spec(D_HID), _row_spec(8),
              _full_spec(D_HID, D_HID), _full_spec(1, D_HID)],
    out_specs=_row_spec(D_HID),
    out_shape=jax.ShapeDtypeStruct((N, D_HID), jnp.float32),
)

_tc3 = pl.pallas_call(
    _tc_mid_body,
    grid=(_GRID,),
    in_specs=[_acc_spec(D_HID), _row_spec(D_HID), _row_spec(8),
              _full_spec(D_HID, D_PAD), _full_spec(1, D_HID)],
    out_specs=_row_spec(D_PAD),
    out_shape=jax.ShapeDtypeStruct((N, D_PAD), jnp.float32),
)

_tc4 = pl.pallas_call(
    _tc4_body,
    grid=(_GRID,),
    in_specs=[_acc_spec(D_PAD), _row_spec(D_PAD), _row_spec(8),
              _full_spec(1, D_PAD)],
    out_specs=pl.BlockSpec((_R, 6), lambda i: (i, 0)),
    out_shape=jax.ShapeDtypeStruct((N, 6), jnp.float32),
)

_sc_deg = _scatter_rows_kernel(D_PAD, 80, 80, with_gather=False)
_sc_hid = _scatter_rows_kernel(D_HID, NB0, NB1, with_gather=True)
_sc_out = _scatter_rows_kernel(D_PAD, 88, 72, with_gather=True)


@jax.jit
def kernel(x, edge_index, W1, b1, W2, b2, W3, b3):
  src = edge_index[0]
  dst = edge_index[1]
  pad = E_PAD - E + NBMAX * B
  srcp = jnp.concatenate([src, jnp.zeros((pad,), jnp.int32)])
  dstp = jnp.concatenate([dst, jnp.full((pad,), N, jnp.int32)])
  srcf = srcp.reshape(NBT + NBMAX, B)
  dstf = dstp.reshape(NBT + NBMAX, B)

  dummy_table = jnp.zeros((1, D_PAD), jnp.float32)
  dega = _sc_deg(dummy_table, srcf, dstf)

  hs1, dis = _tc1(x, W1, dega)
  acc1 = _sc_hid(hs1, srcf, dstf)
  hs2 = _tc_mid(acc1, hs1, dis, W2, b1.reshape(1, D_HID))
  acc2 = _sc_hid(hs2, srcf, dstf)
  W3p = jnp.pad(W3, ((0, 0), (0, D_PAD - W3.shape[1])))
  hs3 = _tc3(acc2, hs2, dis, W3p, b2.reshape(1, D_HID))
  acc3 = _sc_out(hs3, srcf, dstf)
  b3p = jnp.pad(b3, (0, D_PAD - b3.shape[0])).reshape(1, D_PAD)
  return _tc4(acc3, hs3, dis, b3p)


# speedup vs baseline: 1.3563x; 1.3563x over previous
import jax
import jax.numpy as jnp
from jax import lax
from jax.experimental import pallas as pl
from jax.experimental.pallas import tpu as pltpu
from jax.experimental.pallas import tpu_sc as plsc

N = 10000
E = 320000
D_IN = 128
D_HID = 64
D_PAD = 16

NC = 2
NS = 16
NW = NC * NS
B = 80
EPT = 10080
NB = EPT // B
E_PAD = NW * EPT
N_ACC = 10240
ROWS_PER_TILE = N_ACC // NS


def _scatter_rows_kernel(D, with_gather):
  mesh = plsc.VectorSubcoreMesh(core_axis_name="c", subcore_axis_name="s")
  scratch = [
      pltpu.VMEM((NB, B), jnp.int32),
      pltpu.VMEM((NB, B), jnp.int32),
      pltpu.VMEM((B, D), jnp.float32),
      pltpu.VMEM((ROWS_PER_TILE, D), jnp.float32),
      pltpu.VMEM_SHARED((N_ACC, D), jnp.float32),
      pltpu.SemaphoreType.DMA,
  ]

  def body(table, srcr, dstr, out, src_v, dst_v, rows, zbuf, acc, sem):
    cid = lax.axis_index("c")
    sid = lax.axis_index("s")
    wid = cid * NS + sid

    zero16 = jnp.zeros((16,), jnp.float32)

    def zrow(i, carry):
      for j in range(D // 16):
        zbuf[i, pl.ds(j * 16, 16)] = zero16
      return carry

    lax.fori_loop(0, ROWS_PER_TILE, zrow, 0)
    pltpu.sync_copy(zbuf, acc.at[pl.ds(sid * ROWS_PER_TILE, ROWS_PER_TILE)])
    if not with_gather:
      ones16 = jnp.ones((16,), jnp.float32)

      def orow(i, carry):
        for j in range(D // 16):
          rows[i, pl.ds(j * 16, 16)] = ones16
        return carry

      lax.fori_loop(0, B, orow, 0)
    pltpu.sync_copy(srcr.at[wid], src_v)
    pltpu.sync_copy(dstr.at[wid], dst_v)
    plsc.subcore_barrier()

    def edge_batch(j, carry):
      if with_gather:
        pltpu.async_copy(table.at[src_v.at[j]], rows, sem).wait()
      pltpu.sync_copy(rows, acc.at[dst_v.at[j]], add=True)
      return carry

    lax.fori_loop(0, NB, edge_batch, 0)
    plsc.subcore_barrier()
    pltpu.sync_copy(
        acc.at[pl.ds(sid * ROWS_PER_TILE, ROWS_PER_TILE)],
        out.at[cid, pl.ds(sid * ROWS_PER_TILE, ROWS_PER_TILE)])

  return pl.kernel(
      body,
      out_type=jax.ShapeDtypeStruct((NC, N_ACC, D), jnp.float32),
      mesh=mesh,
      scratch_types=scratch,
      compiler_params=pltpu.CompilerParams(use_tc_tiling_on_sc=False),
  )


def _tc1_body(x_ref, w1_ref, dega_ref, hs1_ref, dis_ref):
  deg = dega_ref[0, :, 0:1] + dega_ref[1, :, 0:1] + 1.0
  dis = lax.rsqrt(deg)
  t = jnp.dot(x_ref[...], w1_ref[...], preferred_element_type=jnp.float32)
  hs1_ref[...] = t * dis
  dis_ref[...] = jnp.broadcast_to(dis, dis_ref.shape)


def _tc_mid_body(acc_ref, hs_ref, dis_ref, w_ref, b_ref, out_ref):
  dis = dis_ref[:, 0:1]
  s = acc_ref[0] + acc_ref[1] + hs_ref[...]
  h = jnp.maximum(dis * s + b_ref[...], 0.0)
  out_ref[...] = jnp.dot(h, w_ref[...],
                         preferred_element_type=jnp.float32) * dis


def _tc4_body(acc_ref, hs_ref, dis_ref, b_ref, out_ref):
  dis = dis_ref[:, 0:1]
  o = dis * (acc_ref[0] + acc_ref[1] + hs_ref[...]) + b_ref[...]
  o = o[:, :6]
  m = jnp.max(o, axis=1, keepdims=True)
  e = jnp.exp(o - m)
  lse = jnp.log(jnp.sum(e, axis=1, keepdims=True))
  out_ref[...] = o - m - lse


_R = 1000
_GRID = N // _R


def _row_spec(d):
  return pl.BlockSpec((_R, d), lambda i: (i, 0))


def _acc_spec(d):
  return pl.BlockSpec((NC, _R, d), lambda i: (0, i, 0))


def _full_spec(a, b):
  return pl.BlockSpec((a, b), lambda i: (0, 0))


_tc1 = pl.pallas_call(
    _tc1_body,
    grid=(_GRID,),
    in_specs=[_row_spec(D_IN), _full_spec(D_IN, D_HID), _acc_spec(D_PAD)],
    out_specs=[_row_spec(D_HID), _row_spec(8)],
    out_shape=[jax.ShapeDtypeStruct((N, D_HID), jnp.float32),
               jax.ShapeDtypeStruct((N, 8), jnp.float32)],
)

_tc_mid = pl.pallas_call(
    _tc_mid_body,
    grid=(_GRID,),
    in_specs=[_acc_spec(D_HID), _row_spec(D_HID), _row_spec(8),
              _full_spec(D_HID, D_HID), _full_spec(1, D_HID)],
    out_specs=_row_spec(D_HID),
    out_shape=jax.ShapeDtypeStruct((N, D_HID), jnp.float32),
)

_tc3 = pl.pallas_call(
    _tc_mid_body,
    grid=(_GRID,),
    in_specs=[_acc_spec(D_HID), _row_spec(D_HID), _row_spec(8),
              _full_spec(D_HID, D_PAD), _full_spec(1, D_HID)],
    out_specs=_row_spec(D_PAD),
    out_shape=jax.ShapeDtypeStruct((N, D_PAD), jnp.float32),
)

_tc4 = pl.pallas_call(
    _tc4_body,
    grid=(_GRID,),
    in_specs=[_acc_spec(D_PAD), _row_spec(D_PAD), _row_spec(8),
              _full_spec(1, D_PAD)],
    out_specs=pl.BlockSpec((_R, 6), lambda i: (i, 0)),
    out_shape=jax.ShapeDtypeStruct((N, 6), jnp.float32),
)

_sc_deg = _scatter_rows_kernel(D_PAD, with_gather=False)
_sc_hid = _scatter_rows_kernel(D_HID, with_gather=True)
_sc_out = _scatter_rows_kernel(D_PAD, with_gather=True)


@jax.jit
def kernel(x, edge_index, W1, b1, W2, b2, W3, b3):
  src = edge_index[0]
  dst = edge_index[1]
  pad = E_PAD - E
  srcp = jnp.concatenate([src, jnp.zeros((pad,), jnp.int32)])
  dstp = jnp.concatenate([dst, jnp.full((pad,), N, jnp.int32)])
  srcr = srcp.reshape(NW, NB, B)
  dstr = dstp.reshape(NW, NB, B)

  dummy_table = jnp.zeros((1, D_PAD), jnp.float32)
  dega = _sc_deg(dummy_table, srcr, dstr)

  hs1, dis = _tc1(x, W1, dega)
  acc1 = _sc_hid(hs1, srcr, dstr)
  hs2 = _tc_mid(acc1, hs1, dis, W2, b1.reshape(1, D_HID))
  acc2 = _sc_hid(hs2, srcr, dstr)
  W3p = jnp.pad(W3, ((0, 0), (0, D_PAD - W3.shape[1])))
  hs3 = _tc3(acc2, hs2, dis, W3p, b2.reshape(1, D_HID))
  acc3 = _sc_out(hs3, srcr, dstr)
  b3p = jnp.pad(b3, (0, D_PAD - b3.shape[0])).reshape(1, D_PAD)
  return _tc4(acc3, hs3, dis, b3p)


# speedup vs baseline: 1.5780x; 1.1635x over previous
import jax
import jax.numpy as jnp
from jax import lax
from jax.experimental import pallas as pl
from jax.experimental.pallas import tpu as pltpu
from jax.experimental.pallas import tpu_sc as plsc

N = 10000
E = 320000
D_IN = 128
D_HID = 64
D_PAD = 16

NC = 2
NS = 16
NW = NC * NS
B = 80
NBSUM = 252
NBT = NS * NBSUM
E_PAD = NBT * B
N_ACC = 10240
ROWS_PER_TILE = N_ACC // NS


def _scatter_rows_kernel(D, nb0, nb1, with_gather):
  mesh = plsc.VectorSubcoreMesh(core_axis_name="c", subcore_axis_name="s")
  nbmax = max(nb0, nb1)
  scratch = [
      pltpu.VMEM((nbmax, B), jnp.int32),
      pltpu.VMEM((nbmax, B), jnp.int32),
      pltpu.VMEM((B, D), jnp.float32),
      pltpu.VMEM((B, D), jnp.float32),
      pltpu.VMEM((ROWS_PER_TILE, D), jnp.float32),
      pltpu.VMEM_SHARED((N_ACC, D), jnp.float32),
      pltpu.SemaphoreType.DMA,
      pltpu.SemaphoreType.DMA,
  ]

  def body(table, srcf, dstf, out, src_v, dst_v, rows, rows1, zbuf, acc, sem,
           sem1):
    cid = lax.axis_index("c")
    sid = lax.axis_index("s")
    start_b = lax.select(cid == 0, sid * nb0, NS * nb0 + sid * nb1)

    zero16 = jnp.zeros((16,), jnp.float32)

    def zrow(i, carry):
      for j in range(D // 16):
        zbuf[i, pl.ds(j * 16, 16)] = zero16
      return carry

    lax.fori_loop(0, ROWS_PER_TILE, zrow, 0)
    pltpu.sync_copy(zbuf, acc.at[pl.ds(sid * ROWS_PER_TILE, ROWS_PER_TILE)])
    if not with_gather:
      ones16 = jnp.ones((16,), jnp.float32)

      def orow(i, carry):
        for j in range(D // 16):
          rows[i, pl.ds(j * 16, 16)] = ones16
        return carry

      lax.fori_loop(0, B, orow, 0)
    pltpu.sync_copy(srcf.at[pl.ds(start_b, nbmax)], src_v)
    pltpu.sync_copy(dstf.at[pl.ds(start_b, nbmax)], dst_v)
    plsc.subcore_barrier()

    if with_gather:

      @pl.when(cid == 0)
      def _core0():
        pltpu.async_copy(table.at[src_v.at[0]], rows, sem)

        def pair(t, carry):
          j0 = 2 * t
          j1 = j0 + 1
          pltpu.make_async_copy(table.at[src_v.at[j0]], rows, sem).wait()
          pltpu.async_copy(table.at[src_v.at[j1]], rows1, sem1)
          pltpu.sync_copy(rows, acc.at[dst_v.at[j0]], add=True)
          jn = lax.rem(j0 + 2, nb0)
          pltpu.async_copy(table.at[src_v.at[jn]], rows, sem)
          pltpu.make_async_copy(table.at[src_v.at[j1]], rows1, sem1).wait()
          pltpu.sync_copy(rows1, acc.at[dst_v.at[j1]], add=True)
          return carry

        lax.fori_loop(0, nb0 // 2, pair, 0)
        pltpu.make_async_copy(table.at[src_v.at[0]], rows, sem).wait()

      @pl.when(cid != 0)
      def _core1():
        def edge_batch(j, carry):
          pltpu.async_copy(table.at[src_v.at[j]], rows, sem).wait()
          pltpu.sync_copy(rows, acc.at[dst_v.at[j]], add=True)
          return carry

        lax.fori_loop(0, nb1, edge_batch, 0)
    else:
      nb = lax.select(cid == 0, jnp.int32(nb0), jnp.int32(nb1))

      def ones_batch(j, carry):
        pltpu.sync_copy(rows, acc.at[dst_v.at[j]], add=True)
        return carry

      lax.fori_loop(0, nb, ones_batch, 0)
    plsc.subcore_barrier()
    pltpu.sync_copy(
        acc.at[pl.ds(sid * ROWS_PER_TILE, ROWS_PER_TILE)],
        out.at[cid, pl.ds(sid * ROWS_PER_TILE, ROWS_PER_TILE)])

  return pl.kernel(
      body,
      out_type=jax.ShapeDtypeStruct((NC, N_ACC, D), jnp.float32),
      mesh=mesh,
      scratch_types=scratch,
      compiler_params=pltpu.CompilerParams(use_tc_tiling_on_sc=False),
  )


def _tc1_body(x_ref, w1_ref, dega_ref, hs1_ref, dis_ref):
  deg = dega_ref[0, :, 0:1] + dega_ref[1, :, 0:1] + 1.0
  dis = lax.rsqrt(deg)
  t = jnp.dot(x_ref[...], w1_ref[...], preferred_element_type=jnp.float32)
  hs1_ref[...] = t * dis
  dis_ref[...] = jnp.broadcast_to(dis, dis_ref.shape)


def _tc_mid_body(acc_ref, hs_ref, dis_ref, w_ref, b_ref, out_ref):
  dis = dis_ref[:, 0:1]
  s = acc_ref[0] + acc_ref[1] + hs_ref[...]
  h = jnp.maximum(dis * s + b_ref[...], 0.0)
  out_ref[...] = jnp.dot(h, w_ref[...],
                         preferred_element_type=jnp.float32) * dis


def _tc4_body(acc_ref, hs_ref, dis_ref, b_ref, out_ref):
  dis = dis_ref[:, 0:1]
  o = dis * (acc_ref[0] + acc_ref[1] + hs_ref[...]) + b_ref[...]
  o = o[:, :6]
  m = jnp.max(o, axis=1, keepdims=True)
  e = jnp.exp(o - m)
  lse = jnp.log(jnp.sum(e, axis=1, keepdims=True))
  out_ref[...] = o - m - lse


_R = 1000
_GRID = N // _R


def _row_spec(d):
  return pl.BlockSpec((_R, d), lambda i: (i, 0))


def _acc_spec(d):
  return pl.BlockSpec((NC, _R, d), lambda i: (0, i, 0))


def _full_spec(a, b):
  return pl.BlockSpec((a, b), lambda i: (0, 0))


_tc1 = pl.pallas_call(
    _tc1_body,
    grid=(_GRID,),
    in_specs=[_row_spec(D_IN), _full_spec(D_IN, D_HID), _acc_spec(D_PAD)],
    out_specs=[_row_spec(D_HID), _row_spec(8)],
    out_shape=[jax.ShapeDtypeStruct((N, D_HID), jnp.float32),
               jax.ShapeDtypeStruct((N, 8), jnp.float32)],
)

_tc_mid = pl.pallas_call(
    _tc_mid_body,
    grid=(_GRID,),
    in_specs=[_acc_spec(D_HID), _row_spec(D_HID), _row_spec(8),
              _full_spec(D_HID, D_HID), _full_spec(1, D_HID)],
    out_specs=_row_spec(D_HID),
    out_shape=jax.ShapeDtypeStruct((N, D_HID), jnp.float32),
)

_tc3 = pl.pallas_call(
    _tc_mid_body,
    grid=(_GRID,),
    in_specs=[_acc_spec(D_HID), _row_spec(D_HID), _row_spec(8),
              _full_spec(D_HID, D_PAD), _full_spec(1, D_HID)],
    out_specs=_row_spec(D_PAD),
    out_shape=jax.ShapeDtypeStruct((N, D_PAD), jnp.float32),
)

_tc4 = pl.pallas_call(
    _tc4_body,
    grid=(_GRID,),
    in_specs=[_acc_spec(D_PAD), _row_spec(D_PAD), _row_spec(8),
              _full_spec(1, D_PAD)],
    out_specs=pl.BlockSpec((_R, 6), lambda i: (i, 0)),
    out_shape=jax.ShapeDtypeStruct((N, 6), jnp.float32),
)

NB0_HID = 174
NB1_HID = NBSUM - NB0_HID
NB0_OUT = 132
NB1_OUT = NBSUM - NB0_OUT
_NBMAX = max(NB0_HID, NB0_OUT)

_sc_deg = _scatter_rows_kernel(D_PAD, NBSUM // 2, NBSUM // 2,
                               with_gather=False)
_sc_hid = _scatter_rows_kernel(D_HID, NB0_HID, NB1_HID, with_gather=True)
_sc_out = _scatter_rows_kernel(D_PAD, NB0_OUT, NB1_OUT, with_gather=True)


@jax.jit
def kernel(x, edge_index, W1, b1, W2, b2, W3, b3):
  src = edge_index[0]
  dst = edge_index[1]
  pad = E_PAD - E + _NBMAX * B
  srcp = jnp.concatenate([src, jnp.zeros((pad,), jnp.int32)])
  dstp = jnp.concatenate([dst, jnp.full((pad,), N, jnp.int32)])
  srcf = srcp.reshape(NBT + _NBMAX, B)
  dstf = dstp.reshape(NBT + _NBMAX, B)

  dummy_table = jnp.zeros((1, D_PAD), jnp.float32)
  dega = _sc_deg(dummy_table, srcf, dstf)

  hs1, dis = _tc1(x, W1, dega)
  acc1 = _sc_hid(hs1, srcf, dstf)
  hs2 = _tc_mid(acc1, hs1, dis, W2, b1.reshape(1, D_HID))
  acc2 = _sc_hid(hs2, srcf, dstf)
  W3p = jnp.pad(W3, ((0, 0), (0, D_PAD - W3.shape[1])))
  hs3 = _tc3(acc2, hs2, dis, W3p, b2.reshape(1, D_HID))
  acc3 = _sc_out(hs3, srcf, dstf)
  b3p = jnp.pad(b3, (0, D_PAD - b3.shape[0])).reshape(1, D_PAD)
  return _tc4(acc3, hs3, dis, b3p)


# speedup vs baseline: 1.5822x; 1.0027x over previous
import jax
import jax.numpy as jnp
from jax import lax
from jax.experimental import pallas as pl
from jax.experimental.pallas import tpu as pltpu
from jax.experimental.pallas import tpu_sc as plsc

N = 10000
E = 320000
D_IN = 128
D_HID = 64
D_PAD = 16

NC = 2
NS = 16
NW = NC * NS
B = 80
NBSUM = 252
NBT = NS * NBSUM
E_PAD = NBT * B
N_ACC = 10240
ROWS_PER_TILE = N_ACC // NS


def _scatter_rows_kernel(D, nb0, nb1, with_gather):
  mesh = plsc.VectorSubcoreMesh(core_axis_name="c", subcore_axis_name="s")
  nbmax = max(nb0, nb1)
  scratch = [
      pltpu.VMEM((nbmax, B), jnp.int32),
      pltpu.VMEM((nbmax, B), jnp.int32),
      pltpu.VMEM((B, D), jnp.float32),
      pltpu.VMEM((B, D), jnp.float32),
      pltpu.VMEM((ROWS_PER_TILE, D), jnp.float32),
      pltpu.VMEM_SHARED((N_ACC, D), jnp.float32),
      pltpu.SemaphoreType.DMA,
      pltpu.SemaphoreType.DMA,
  ]

  def body(table, srcf, dstf, out, src_v, dst_v, rows, rows1, zbuf, acc, sem,
           sem1):
    cid = lax.axis_index("c")
    sid = lax.axis_index("s")
    start_b = lax.select(cid == 0, sid * nb0, NS * nb0 + sid * nb1)

    zero16 = jnp.zeros((16,), jnp.float32)

    def zrow(i, carry):
      for j in range(D // 16):
        zbuf[i, pl.ds(j * 16, 16)] = zero16
      return carry

    lax.fori_loop(0, ROWS_PER_TILE, zrow, 0)
    pltpu.sync_copy(zbuf, acc.at[pl.ds(sid * ROWS_PER_TILE, ROWS_PER_TILE)])
    if not with_gather:
      ones16 = jnp.ones((16,), jnp.float32)

      def orow(i, carry):
        for j in range(D // 16):
          rows[i, pl.ds(j * 16, 16)] = ones16
        return carry

      lax.fori_loop(0, B, orow, 0)
    pltpu.sync_copy(srcf.at[pl.ds(start_b, nbmax)], src_v)
    pltpu.sync_copy(dstf.at[pl.ds(start_b, nbmax)], dst_v)
    plsc.subcore_barrier()

    if with_gather:

      @pl.when(cid == 0)
      def _core0():
        pltpu.async_copy(table.at[src_v.at[0]], rows, sem)

        def pair(t, carry):
          j0 = 2 * t
          j1 = j0 + 1
          pltpu.make_async_copy(table.at[src_v.at[j0]], rows, sem).wait()
          pltpu.async_copy(table.at[src_v.at[j1]], rows1, sem1)
          pltpu.sync_copy(rows, acc.at[dst_v.at[j0]], add=True)
          jn = lax.rem(j0 + 2, nb0)
          pltpu.async_copy(table.at[src_v.at[jn]], rows, sem)
          pltpu.make_async_copy(table.at[src_v.at[j1]], rows1, sem1).wait()
          pltpu.sync_copy(rows1, acc.at[dst_v.at[j1]], add=True)
          return carry

        lax.fori_loop(0, nb0 // 2, pair, 0)
        pltpu.make_async_copy(table.at[src_v.at[0]], rows, sem).wait()

      @pl.when(cid != 0)
      def _core1():
        def edge_batch(j, carry):
          pltpu.async_copy(table.at[src_v.at[j]], rows, sem).wait()
          pltpu.sync_copy(rows, acc.at[dst_v.at[j]], add=True)
          return carry

        lax.fori_loop(0, nb1, edge_batch, 0)
    else:
      nb = lax.select(cid == 0, jnp.int32(nb0), jnp.int32(nb1))

      def ones_batch(j, carry):
        pltpu.sync_copy(rows, acc.at[dst_v.at[j]], add=True)
        return carry

      lax.fori_loop(0, nb, ones_batch, 0)
    plsc.subcore_barrier()
    pltpu.sync_copy(
        acc.at[pl.ds(sid * ROWS_PER_TILE, ROWS_PER_TILE)],
        out.at[cid, pl.ds(sid * ROWS_PER_TILE, ROWS_PER_TILE)])

  return pl.kernel(
      body,
      out_type=jax.ShapeDtypeStruct((NC, N_ACC, D), jnp.float32),
      mesh=mesh,
      scratch_types=scratch,
      compiler_params=pltpu.CompilerParams(use_tc_tiling_on_sc=False),
  )


def _tc1_body(x_ref, w1_ref, dega_ref, hs1_ref, dis_ref):
  deg = dega_ref[0, :, 0:1] + dega_ref[1, :, 0:1] + 1.0
  dis = lax.rsqrt(deg)
  t = jnp.dot(x_ref[...], w1_ref[...], preferred_element_type=jnp.float32)
  hs1_ref[...] = t * dis
  dis_ref[...] = jnp.broadcast_to(dis, dis_ref.shape)


def _tc_mid_body(acc_ref, hs_ref, dis_ref, w_ref, b_ref, out_ref):
  dis = dis_ref[:, 0:1]
  s = acc_ref[0] + acc_ref[1] + hs_ref[...]
  h = jnp.maximum(dis * s + b_ref[...], 0.0)
  out_ref[...] = jnp.dot(h, w_ref[...],
                         preferred_element_type=jnp.float32) * dis


def _tc4_body(acc_ref, hs_ref, dis_ref, b_ref, out_ref):
  dis = dis_ref[:, 0:1]
  o = dis * (acc_ref[0] + acc_ref[1] + hs_ref[...]) + b_ref[...]
  o = o[:, :6]
  m = jnp.max(o, axis=1, keepdims=True)
  e = jnp.exp(o - m)
  lse = jnp.log(jnp.sum(e, axis=1, keepdims=True))
  out_ref[...] = o - m - lse


_R = 1000
_GRID = N // _R


def _row_spec(d):
  return pl.BlockSpec((_R, d), lambda i: (i, 0))


def _acc_spec(d):
  return pl.BlockSpec((NC, _R, d), lambda i: (0, i, 0))


def _full_spec(a, b):
  return pl.BlockSpec((a, b), lambda i: (0, 0))


_tc1 = pl.pallas_call(
    _tc1_body,
    grid=(_GRID,),
    in_specs=[_row_spec(D_IN), _full_spec(D_IN, D_HID), _acc_spec(D_PAD)],
    out_specs=[_row_spec(D_HID), _row_spec(8)],
    out_shape=[jax.ShapeDtypeStruct((N, D_HID), jnp.float32),
               jax.ShapeDtypeStruct((N, 8), jnp.float32)],
)

_tc_mid = pl.pallas_call(
    _tc_mid_body,
    grid=(_GRID,),
    in_specs=[_acc_spec(D_HID), _row_spec(D_HID), _row_spec(8),
              _full_spec(D_HID, D_HID), _full_spec(1, D_HID)],
    out_specs=_row_spec(D_HID),
    out_shape=jax.ShapeDtypeStruct((N, D_HID), jnp.float32),
)

_tc3 = pl.pallas_call(
    _tc_mid_body,
    grid=(_GRID,),
    in_specs=[_acc_spec(D_HID), _row_spec(D_HID), _row_spec(8),
              _full_spec(D_HID, D_PAD), _full_spec(1, D_HID)],
    out_specs=_row_spec(D_PAD),
    out_shape=jax.ShapeDtypeStruct((N, D_PAD), jnp.float32),
)

_tc4 = pl.pallas_call(
    _tc4_body,
    grid=(_GRID,),
    in_specs=[_acc_spec(D_PAD), _row_spec(D_PAD), _row_spec(8),
              _full_spec(1, D_PAD)],
    out_specs=pl.BlockSpec((_R, 6), lambda i: (i, 0)),
    out_shape=jax.ShapeDtypeStruct((N, 6), jnp.float32),
)

NB0_HID = 160
NB1_HID = NBSUM - NB0_HID
NB0_OUT = 164
NB1_OUT = NBSUM - NB0_OUT
_NBMAX = max(NB0_HID, NB0_OUT)

_sc_deg = _scatter_rows_kernel(D_PAD, NBSUM // 2, NBSUM // 2,
                               with_gather=False)
_sc_hid = _scatter_rows_kernel(D_HID, NB0_HID, NB1_HID, with_gather=True)
_sc_out = _scatter_rows_kernel(D_PAD, NB0_OUT, NB1_OUT, with_gather=True)


@jax.jit
def kernel(x, edge_index, W1, b1, W2, b2, W3, b3):
  src = edge_index[0]
  dst = edge_index[1]
  pad = E_PAD - E + _NBMAX * B
  srcp = jnp.concatenate([src, jnp.zeros((pad,), jnp.int32)])
  dstp = jnp.concatenate([dst, jnp.full((pad,), N, jnp.int32)])
  srcf = srcp.reshape(NBT + _NBMAX, B)
  dstf = dstp.reshape(NBT + _NBMAX, B)

  dummy_table = jnp.zeros((1, D_PAD), jnp.float32)
  dega = _sc_deg(dummy_table, srcf, dstf)

  hs1, dis = _tc1(x, W1, dega)
  acc1 = _sc_hid(hs1, srcf, dstf)
  hs2 = _tc_mid(acc1, hs1, dis, W2, b1.reshape(1, D_HID))
  acc2 = _sc_hid(hs2, srcf, dstf)
  W3p = jnp.pad(W3, ((0, 0), (0, D_PAD - W3.shape[1])))
  hs3 = _tc3(acc2, hs2, dis, W3p, b2.reshape(1, D_HID))
  acc3 = _sc_out(hs3, srcf, dstf)
  b3p = jnp.pad(b3, (0, D_PAD - b3.shape[0])).reshape(1, D_PAD)
  return _tc4(acc3, hs3, dis, b3p)
